# baseline (speedup 1.0000x reference)
import functools

import jax
import jax.numpy as jnp
from jax import lax
from jax.experimental import pallas as pl
from jax.experimental.pallas import tpu as pltpu
from jax.experimental.pallas import tpu_sc as plsc

_B = 16384
_D = 16
_NC, _NS = 2, 16
_NW = _NC * _NS
_BPW = _B // _NW
_N = 1000000
_NCOLS = 7813
_TAIL = _NCOLS - 1
_L = 16
_CHUNKS = _BPW // _L


def _detile_body(zt_hbm, ztail_hbm, zst_hbm, sem):
    wid = lax.axis_index("s") * _NC + lax.axis_index("c")

    def step(c, carry, *, wait):
        @pl.when(c != _TAIL)
        def _():
            cp = pltpu.make_async_copy(
                zt_hbm.at[:, pl.ds(c * 128, 128)], zst_hbm.at[c], sem
            )
            cp.wait() if wait else cp.start()

        @pl.when(c == _TAIL)
        def _():
            cp = pltpu.make_async_copy(ztail_hbm, zst_hbm.at[_TAIL], sem)
            cp.wait() if wait else cp.start()

        return carry

    def col(k, body):
        return wid + k * _NW

    nit = (_NCOLS - wid + _NW - 1) // _NW
    lax.fori_loop(
        0, nit, lambda k, a: step(wid + k * _NW, a, wait=False), 0, unroll=False
    )
    lax.fori_loop(
        0, nit, lambda k, a: step(wid + k * _NW, a, wait=True), 0, unroll=False
    )


def _gather_body(zst_hbm, idx_hbm, out_hbm, idx_v, off_v, val_v, isem, gsem):
    wid = lax.axis_index("s") * _NC + lax.axis_index("c")
    base = wid * _BPW
    cp = pltpu.make_async_copy(idx_hbm.at[pl.ds(base, _BPW)], idx_v, isem)
    cp.start()
    cp.wait()

    def compute(k, carry):
        idx = idx_v[pl.ds(k * _L, _L)]
        c = lax.shift_right_logical(idx, 7)
        il = lax.bitwise_and(idx, 127)
        boff = c * 2048 + il
        for j in range(_D):
            off_v[j, pl.ds(k * _L, _L)] = boff + (j * 128)
        return carry

    lax.fori_loop(0, _CHUNKS, compute, 0, unroll=False)

    for j in range(_D):
        pltpu.make_async_copy(
            zst_hbm.at[off_v.at[j]], val_v.at[j], gsem
        ).start()
    for j in range(_D):
        pltpu.make_async_copy(
            zst_hbm.at[off_v.at[j]], val_v.at[j], gsem
        ).wait()

    for j in range(_D):
        pltpu.sync_copy(val_v.at[j], out_hbm.at[j, pl.ds(base, _BPW)])


def kernel(indices, z):
    mesh = plsc.VectorSubcoreMesh(core_axis_name="c", subcore_axis_name="s")
    detile = pl.kernel(
        _detile_body,
        mesh=mesh,
        out_type=jax.ShapeDtypeStruct((_NCOLS, _D, 128), jnp.float32),
        scratch_types=[pltpu.SemaphoreType.DMA],
    )
    gather = pl.kernel(
        _gather_body,
        mesh=mesh,
        out_type=jax.ShapeDtypeStruct((_D, _B), jnp.float32),
        scratch_types=[
            pltpu.VMEM((_BPW,), jnp.int32),
            pltpu.VMEM((_D, _BPW), jnp.int32),
            pltpu.VMEM((_D, _BPW), jnp.float32),
            pltpu.SemaphoreType.DMA,
            pltpu.SemaphoreType.DMA,
        ],
        compiler_params=pltpu.CompilerParams(use_tc_tiling_on_sc=False),
    )
    zt = z.T
    ztail = jnp.pad(zt[:, _TAIL * 128 :], ((0, 0), (0, _NCOLS * 128 - _N)))
    zst = detile(zt, ztail)
    zst_flat = zst.reshape(-1)
    out_t = gather(zst_flat, indices.astype(jnp.int32))
    return out_t.T


# speedup vs baseline: 21.6856x; 21.6856x over previous
import functools

import jax
import jax.numpy as jnp
from jax import lax
from jax.experimental import pallas as pl
from jax.experimental.pallas import tpu as pltpu
from jax.experimental.pallas import tpu_sc as plsc

_B = 16384
_D = 16
_NC, _NS = 2, 16
_NW = _NC * _NS
_BPW = _B // _NW
_N = 1000000
_NCOLS = 7813
_TAIL = _NCOLS - 1
_L = 16
_CHUNKS = _BPW // _L


_NBUF = 8
_LAG = 4


def _detile_body(zt_hbm, ztail_hbm, zst_hbm, bufs, in_sem, out_sem):
    wid = lax.axis_index("s") * _NC + lax.axis_index("c")
    nit = (_NCOLS - wid + _NW - 1) // _NW

    def in_copy(g):
        c = wid + g * _NW
        buf = bufs.at[lax.rem(g, _NBUF)]

        @pl.when(c != _TAIL)
        def _():
            pltpu.make_async_copy(
                zt_hbm.at[:, pl.ds(c * 128, 128)], buf, in_sem
            ).start()

        @pl.when(c == _TAIL)
        def _():
            pltpu.make_async_copy(ztail_hbm, buf, in_sem).start()

    def in_wait(g):
        pltpu.make_async_copy(
            zt_hbm.at[:, pl.ds(0, 128)], bufs.at[lax.rem(g, _NBUF)], in_sem
        ).wait()

    def out_copy(g):
        c = wid + g * _NW
        pltpu.make_async_copy(
            bufs.at[lax.rem(g, _NBUF)], zst_hbm.at[c], out_sem
        ).start()

    def out_wait(g):
        pltpu.make_async_copy(
            bufs.at[lax.rem(g, _NBUF)], zst_hbm.at[wid], out_sem
        ).wait()

    def step(g, carry):
        @pl.when(g >= _NBUF)
        def _():
            out_wait(g - _NBUF)

        @pl.when(g < nit)
        def _():
            in_copy(g)

        @pl.when(jnp.logical_and(g >= _LAG, g - _LAG < nit))
        def _():
            in_wait(g - _LAG)
            out_copy(g - _LAG)

        return carry

    lax.fori_loop(0, nit + _LAG, step, 0, unroll=False)

    def drain(k, carry):
        @pl.when(k < jnp.minimum(nit, _NBUF - _LAG))
        def _():
            out_wait(0)

        return carry

    lax.fori_loop(0, _NBUF - _LAG, drain, 0, unroll=False)


def _gather_body(zst_hbm, idx_hbm, out_hbm, idx_v, off_v, val_v, isem, gsem):
    wid = lax.axis_index("s") * _NC + lax.axis_index("c")
    base = wid * _BPW
    cp = pltpu.make_async_copy(idx_hbm.at[pl.ds(base, _BPW)], idx_v, isem)
    cp.start()
    cp.wait()

    def compute(k, carry):
        idx = idx_v[pl.ds(k * _L, _L)]
        c = lax.shift_right_logical(idx, 7)
        il = lax.bitwise_and(idx, 127)
        boff = c * 2048 + il
        for j in range(_D):
            off_v[j, pl.ds(k * _L, _L)] = boff + (j * 128)
        return carry

    lax.fori_loop(0, _CHUNKS, compute, 0, unroll=False)

    for j in range(_D):
        pltpu.make_async_copy(
            zst_hbm.at[off_v.at[j]], val_v.at[j], gsem
        ).start()
    for j in range(_D):
        pltpu.make_async_copy(
            zst_hbm.at[off_v.at[j]], val_v.at[j], gsem
        ).wait()

    for j in range(_D):
        pltpu.sync_copy(val_v.at[j], out_hbm.at[j, pl.ds(base, _BPW)])


def kernel(indices, z):
    mesh = plsc.VectorSubcoreMesh(core_axis_name="c", subcore_axis_name="s")
    detile = pl.kernel(
        _detile_body,
        mesh=mesh,
        out_type=jax.ShapeDtypeStruct((_NCOLS, _D, 128), jnp.float32),
        scratch_types=[
            pltpu.VMEM((_NBUF, _D, 128), jnp.float32),
            pltpu.SemaphoreType.DMA,
            pltpu.SemaphoreType.DMA,
        ],
    )
    gather = pl.kernel(
        _gather_body,
        mesh=mesh,
        out_type=jax.ShapeDtypeStruct((_D, _B), jnp.float32),
        scratch_types=[
            pltpu.VMEM((_BPW,), jnp.int32),
            pltpu.VMEM((_D, _BPW), jnp.int32),
            pltpu.VMEM((_D, _BPW), jnp.float32),
            pltpu.SemaphoreType.DMA,
            pltpu.SemaphoreType.DMA,
        ],
        compiler_params=pltpu.CompilerParams(use_tc_tiling_on_sc=False),
    )
    zt = z.T
    ztail = jnp.pad(zt[:, _TAIL * 128 :], ((0, 0), (0, _NCOLS * 128 - _N)))
    zst = detile(zt, ztail)
    zst_flat = zst.reshape(-1)
    out_t = gather(zst_flat, indices.astype(jnp.int32))
    return out_t.T


# speedup vs baseline: 24.2512x; 1.1183x over previous
import functools

import jax
import jax.numpy as jnp
from jax import lax
from jax.experimental import pallas as pl
from jax.experimental.pallas import tpu as pltpu
from jax.experimental.pallas import tpu_sc as plsc

_B = 16384
_D = 16
_NC, _NS = 2, 16
_NW = _NC * _NS
_BPW = _B // _NW
_N = 1000000
_NCOLS = 7813
_TAIL = _NCOLS - 1
_L = 16
_CHUNKS = _BPW // _L


_NBUF = 16
_LAG = 8


def _detile_body(zt_hbm, ztail_hbm, zst_hbm, bufs, in_sem, out_sem):
    wid = lax.axis_index("s") * _NC + lax.axis_index("c")
    nit = (_NCOLS - wid + _NW - 1) // _NW

    def in_copy(g):
        c = wid + g * _NW
        buf = bufs.at[lax.rem(g, _NBUF)]

        @pl.when(c != _TAIL)
        def _():
            pltpu.make_async_copy(
                zt_hbm.at[:, pl.ds(c * 128, 128)], buf, in_sem
            ).start()

        @pl.when(c == _TAIL)
        def _():
            pltpu.make_async_copy(ztail_hbm, buf, in_sem).start()

    def in_wait(g):
        pltpu.make_async_copy(
            zt_hbm.at[:, pl.ds(0, 128)], bufs.at[lax.rem(g, _NBUF)], in_sem
        ).wait()

    def out_copy(g):
        c = wid + g * _NW
        pltpu.make_async_copy(
            bufs.at[lax.rem(g, _NBUF)], zst_hbm.at[c], out_sem
        ).start()

    def out_wait(g):
        pltpu.make_async_copy(
            bufs.at[lax.rem(g, _NBUF)], zst_hbm.at[wid], out_sem
        ).wait()

    def step(g, carry):
        @pl.when(g >= _NBUF)
        def _():
            out_wait(g - _NBUF)

        @pl.when(g < nit)
        def _():
            in_copy(g)

        @pl.when(jnp.logical_and(g >= _LAG, g - _LAG < nit))
        def _():
            in_wait(g - _LAG)
            out_copy(g - _LAG)

        return carry

    lax.fori_loop(0, nit + _LAG, step, 0, unroll=False)

    def drain(k, carry):
        @pl.when(k < jnp.minimum(nit, _NBUF - _LAG))
        def _():
            out_wait(0)

        return carry

    lax.fori_loop(0, _NBUF - _LAG, drain, 0, unroll=False)


def _gather_body(zst_hbm, idx_hbm, out_hbm, idx_v, off_v, val_v, isem, gsem):
    wid = lax.axis_index("s") * _NC + lax.axis_index("c")
    base = wid * _BPW
    cp = pltpu.make_async_copy(idx_hbm.at[pl.ds(base, _BPW)], idx_v, isem)
    cp.start()
    cp.wait()

    def compute(k, carry):
        idx = idx_v[pl.ds(k * _L, _L)]
        c = lax.shift_right_logical(idx, 7)
        il = lax.bitwise_and(idx, 127)
        boff = c * 2048 + il
        for j in range(_D):
            off_v[j, pl.ds(k * _L, _L)] = boff + (j * 128)
        return carry

    lax.fori_loop(0, _CHUNKS, compute, 0, unroll=False)

    for j in range(_D):
        pltpu.make_async_copy(
            zst_hbm.at[off_v.at[j]], val_v.at[j], gsem
        ).start()
    for j in range(_D):
        pltpu.make_async_copy(
            zst_hbm.at[off_v.at[j]], val_v.at[j], gsem
        ).wait()

    for j in range(_D):
        pltpu.sync_copy(val_v.at[j], out_hbm.at[j, pl.ds(base, _BPW)])


def kernel(indices, z):
    mesh = plsc.VectorSubcoreMesh(core_axis_name="c", subcore_axis_name="s")
    detile = pl.kernel(
        _detile_body,
        mesh=mesh,
        out_type=jax.ShapeDtypeStruct((_NCOLS, _D, 128), jnp.float32),
        scratch_types=[
            pltpu.VMEM((_NBUF, _D, 128), jnp.float32),
            pltpu.SemaphoreType.DMA,
            pltpu.SemaphoreType.DMA,
        ],
    )
    gather = pl.kernel(
        _gather_body,
        mesh=mesh,
        out_type=jax.ShapeDtypeStruct((_D, _B), jnp.float32),
        scratch_types=[
            pltpu.VMEM((_BPW,), jnp.int32),
            pltpu.VMEM((_D, _BPW), jnp.int32),
            pltpu.VMEM((_D, _BPW), jnp.float32),
            pltpu.SemaphoreType.DMA,
            pltpu.SemaphoreType.DMA,
        ],
        compiler_params=pltpu.CompilerParams(use_tc_tiling_on_sc=False),
    )
    zt = z.T
    ztail = jnp.pad(zt[:, _TAIL * 128 :], ((0, 0), (0, _NCOLS * 128 - _N)))
    zst = detile(zt, ztail)
    zst_flat = zst.reshape(-1)
    out_t = gather(zst_flat, indices.astype(jnp.int32))
    return out_t.T
